# baseline (speedup 1.0000x reference)
import dataclasses
import functools

import jax
import jax.numpy as jnp
from jax import lax
from jax.experimental import pallas as pl
from jax.experimental.pallas import tpu as pltpu
from jax.experimental.pallas import tpu_sc as plsc

N = 10000
E = 320000
D = 128

NC = 2
NS = 16
L = 16
NW = NC * NS

NPAD = 10240
NPT = NPAD // NW
CE = 4000
NCH = E // CE
RG = 128

BM = 1280


def _mm_body(x_ref, w_ref, b_ref, p_ref, q_ref):
    w1 = w_ref[:, :D]
    w2 = w_ref[:, D:]
    xb = x_ref[...]
    dn = (((1,), (1,)), ((), ()))
    p_ref[...] = lax.dot_general(xb, w1 - w2, dn,
                                 preferred_element_type=jnp.float32) + b_ref[...]
    q_ref[...] = lax.dot_general(xb, w2, dn, preferred_element_type=jnp.float32)


def _matmul_stage(x_pad, W, b):
    return pl.pallas_call(
        _mm_body,
        grid=(NPAD // BM,),
        in_specs=[
            pl.BlockSpec((BM, D), lambda i: (i, 0)),
            pl.BlockSpec((D, 2 * D), lambda i: (0, 0)),
            pl.BlockSpec((1, D), lambda i: (0, 0)),
        ],
        out_specs=[
            pl.BlockSpec((BM, D), lambda i: (i, 0)),
            pl.BlockSpec((BM, D), lambda i: (i, 0)),
        ],
        out_shape=[jax.ShapeDtypeStruct((NPAD, D), jnp.float32)] * 2,
    )(x_pad, W, b.reshape(1, D))


def _sc_compiler_params():
    cp = pltpu.CompilerParams()
    if "needs_layout_passes" in pltpu.CompilerParams.__dataclass_fields__:
        cp = dataclasses.replace(cp, needs_layout_passes=False)
    return cp


def _sc_body(p_hbm, q_hbm, src_hbm, dst_hbm, out_hbm,
             table, dstbuf, srcbuf, moff, msrc, rows, prows, sem):
    wid = lax.axis_index("s") * NC + lax.axis_index("c")
    lo = wid * NPT

    neg_inf = jnp.full((L,), -jnp.inf, dtype=jnp.float32)
    zeros_i = jnp.zeros((L,), jnp.int32)

    @pl.loop(0, NPT + 1)
    def _init_table(r):
        for c in range(D // L):
            table[r, pl.ds(c * L, L)] = neg_inf

    @pl.loop(0, CE // L)
    def _init_msrc(g):
        msrc[pl.ds(g * L, L)] = zeros_i

    @pl.loop(0, NCH)
    def _chunk(ch):
        base = ch * CE
        pltpu.sync_copy(dst_hbm.at[pl.ds(base, CE)], dstbuf)
        pltpu.sync_copy(src_hbm.at[pl.ds(base, CE)], srcbuf)

        def _filter(g, cnt):
            dv = dstbuf[pl.ds(g * L, L)]
            sv = srcbuf[pl.ds(g * L, L)]
            m = (dv >= lo) & (dv < lo + NPT)
            pref = plsc.cumsum(m.astype(jnp.int32))
            pos = cnt + pref - 1
            plsc.store_scatter(msrc, [pos], sv, mask=m)
            plsc.store_scatter(moff, [pos], dv - lo, mask=m)
            return cnt + jnp.max(pref)

        cnt = lax.fori_loop(0, CE // L, _filter, jnp.int32(0))

        padend = (cnt + (L - 1)) & ~(L - 1)
        iot = lax.iota(jnp.int32, L)
        plsc.store_scatter(moff, [cnt + iot],
                           jnp.full((L,), NPT, jnp.int32),
                           mask=(cnt + iot) < padend)

        nb = (cnt + (RG - 1)) >> 7

        def _gather_batch(bi, carry):
            gbase = bi * RG
            pltpu.async_copy(q_hbm.at[msrc.at[pl.ds(gbase, RG)]], rows, sem).wait()
            nvalid = jnp.minimum(cnt - gbase, RG)
            ng = (nvalid + (L - 1)) >> 4

            def _update(g2, c2):
                dlocv = moff[pl.ds(gbase + g2 * L, L)]
                for k in range(L):
                    dloc = dlocv[k]
                    j = g2 * L + k
                    for c in range(D // L):
                        sl = pl.ds(c * L, L)
                        table[dloc, sl] = jnp.maximum(table[dloc, sl],
                                                      rows[j, sl])
                return c2

            return lax.fori_loop(0, ng, _update, carry)

        lax.fori_loop(0, nb, _gather_batch, jnp.int32(0))

    pltpu.sync_copy(p_hbm.at[pl.ds(lo, NPT)], prows)

    @pl.loop(0, NPT)
    def _finish(r):
        for c in range(D // L):
            sl = pl.ds(c * L, L)
            table[r, sl] = jnp.maximum(table[r, sl] + prows[r, sl], 0.0)

    pltpu.sync_copy(table.at[pl.ds(0, NPT)], out_hbm.at[pl.ds(lo, NPT)])


_segment_max_stage = functools.partial(
    pl.kernel,
    out_type=jax.ShapeDtypeStruct((NPAD, D), jnp.float32),
    mesh=plsc.VectorSubcoreMesh(core_axis_name="c", subcore_axis_name="s"),
    scratch_types=[
        pltpu.VMEM((NPT + 1, D), jnp.float32),
        pltpu.VMEM((CE,), jnp.int32),
        pltpu.VMEM((CE,), jnp.int32),
        pltpu.VMEM((CE,), jnp.int32),
        pltpu.VMEM((CE,), jnp.int32),
        pltpu.VMEM((RG, D), jnp.float32),
        pltpu.VMEM((NPT, D), jnp.float32),
        pltpu.SemaphoreType.DMA,
    ],
    compiler_params=_sc_compiler_params(),
)(_sc_body)


def kernel(x, edge_index, W, b):
    x_pad = jnp.pad(x, ((0, NPAD - N), (0, 0)))
    P, Q = _matmul_stage(x_pad, W, b)
    out = _segment_max_stage(P, Q, edge_index[0], edge_index[1])
    return out[:N]


# speedup vs baseline: 3.8875x; 3.8875x over previous
import dataclasses
import functools

import jax
import jax.numpy as jnp
from jax import lax
from jax.experimental import pallas as pl
from jax.experimental.pallas import tpu as pltpu
from jax.experimental.pallas import tpu_sc as plsc

N = 10000
E = 320000
D = 128

NC = 2
NS = 16
L = 16
NW = NC * NS

NPAD = 10240
NPT = NPAD // NW
CE = 16000
NCH = E // CE
RG = 128

BM = 1280


def _mm_body(x_ref, w_ref, b_ref, p_ref, q_ref):
    w1 = w_ref[:, :D]
    w2 = w_ref[:, D:]
    xb = x_ref[...]
    dn = (((1,), (1,)), ((), ()))
    p_ref[...] = lax.dot_general(xb, w1 - w2, dn,
                                 preferred_element_type=jnp.float32) + b_ref[...]
    q_ref[...] = lax.dot_general(xb, w2, dn, preferred_element_type=jnp.float32)


def _matmul_stage(x_pad, W, b):
    return pl.pallas_call(
        _mm_body,
        grid=(NPAD // BM,),
        in_specs=[
            pl.BlockSpec((BM, D), lambda i: (i, 0)),
            pl.BlockSpec((D, 2 * D), lambda i: (0, 0)),
            pl.BlockSpec((1, D), lambda i: (0, 0)),
        ],
        out_specs=[
            pl.BlockSpec((BM, D), lambda i: (i, 0)),
            pl.BlockSpec((BM, D), lambda i: (i, 0)),
        ],
        out_shape=[jax.ShapeDtypeStruct((NPAD, D), jnp.float32)] * 2,
    )(x_pad, W, b.reshape(1, D))


def _sc_compiler_params():
    cp = pltpu.CompilerParams()
    if "needs_layout_passes" in pltpu.CompilerParams.__dataclass_fields__:
        cp = dataclasses.replace(cp, needs_layout_passes=False)
    return cp


NEP = 64


def _sc_body(p_hbm, q_hbm, src_hbm, dst_hbm, out_hbm,
             table, dstbuf, srcbuf, moff, msrc, rows, sem):
    wid = lax.axis_index("s") * NC + lax.axis_index("c")
    lo = wid * NPT

    neg_inf = jnp.full((L,), -jnp.inf, dtype=jnp.float32)
    zeros_i = jnp.zeros((L,), jnp.int32)

    @pl.loop(0, NPT + 1)
    def _init_table(r):
        for c in range(D // L):
            table[r, pl.ds(c * L, L)] = neg_inf

    @pl.loop(0, CE // L)
    def _init_msrc(g):
        msrc[pl.ds(g * L, L)] = zeros_i

    @pl.loop(0, NCH)
    def _chunk(ch):
        base = ch * CE
        pltpu.sync_copy(dst_hbm.at[pl.ds(base, CE)], dstbuf)
        pltpu.sync_copy(src_hbm.at[pl.ds(base, CE)], srcbuf)

        def _filter(g, cnt):
            dv = dstbuf[pl.ds(g * L, L)]
            sv = srcbuf[pl.ds(g * L, L)]
            m = (dv >= lo) & (dv < lo + NPT)
            plsc.store_compressed(msrc.at[pl.ds(cnt, L)], sv, mask=m)
            plsc.store_compressed(moff.at[pl.ds(cnt, L)], dv - lo, mask=m)
            nm = plsc.all_reduce_population_count(m)
            return cnt + nm[0]

        cnt = lax.fori_loop(0, CE // L, _filter, jnp.int32(0))

        padend = (cnt + (L - 1)) & ~(L - 1)
        iot = lax.iota(jnp.int32, L)
        plsc.store_scatter(moff, [cnt + iot],
                           jnp.full((L,), NPT, jnp.int32),
                           mask=(cnt + iot) < padend)

        nb = (cnt + (RG - 1)) >> 7

        def _gather_batch(bi, carry):
            gbase = bi * RG
            pltpu.async_copy(q_hbm.at[msrc.at[pl.ds(gbase, RG)]], rows, sem).wait()
            nvalid = jnp.minimum(cnt - gbase, RG)
            ng = (nvalid + (L - 1)) >> 4

            def _update(g2, c2):
                dlocv = moff[pl.ds(gbase + g2 * L, L)]
                for k in range(L):
                    dloc = dlocv[k]
                    j = g2 * L + k
                    for c in range(D // L):
                        sl = pl.ds(c * L, L)
                        table[dloc, sl] = jnp.maximum(table[dloc, sl],
                                                      rows[j, sl])
                return c2

            return lax.fori_loop(0, ng, _update, carry)

        lax.fori_loop(0, nb, _gather_batch, jnp.int32(0))

    @pl.loop(0, NPT // NEP)
    def _finish(s):
        pltpu.sync_copy(p_hbm.at[pl.ds(lo + s * NEP, NEP)],
                        rows.at[pl.ds(0, NEP)])

        @pl.loop(0, NEP)
        def _finish_row(r):
            for c in range(D // L):
                sl = pl.ds(c * L, L)
                table[s * NEP + r, sl] = jnp.maximum(
                    table[s * NEP + r, sl] + rows[r, sl], 0.0)

    pltpu.sync_copy(table.at[pl.ds(0, NPT)], out_hbm.at[pl.ds(lo, NPT)])


_segment_max_stage = functools.partial(
    pl.kernel,
    out_type=jax.ShapeDtypeStruct((NPAD, D), jnp.float32),
    mesh=plsc.VectorSubcoreMesh(core_axis_name="c", subcore_axis_name="s"),
    scratch_types=[
        pltpu.VMEM((NPT + 1, D), jnp.float32),
        pltpu.VMEM((CE,), jnp.int32),
        pltpu.VMEM((CE,), jnp.int32),
        pltpu.VMEM((CE,), jnp.int32),
        pltpu.VMEM((CE,), jnp.int32),
        pltpu.VMEM((RG, D), jnp.float32),
        pltpu.SemaphoreType.DMA,
    ],
    compiler_params=_sc_compiler_params(),
)(_sc_body)


def kernel(x, edge_index, W, b):
    x_pad = jnp.pad(x, ((0, NPAD - N), (0, 0)))
    P, Q = _matmul_stage(x_pad, W, b)
    out = _segment_max_stage(P, Q, edge_index[0], edge_index[1])
    return out[:N]
